# Initial kernel scaffold; baseline (speedup 1.0000x reference)
import jax
import jax.numpy as jnp
from jax.experimental import pallas as pl


def kernel(x_one, edge_index_one, edge_attr_emb_one, x_two, edge_index_two, edge_attr_emb_two, W1, We1, as1, ad1, ae1, b1, W2, We2, as2, ad2, ae2, b2):
    raise NotImplementedError("write your pallas kernel here")


# speedup vs baseline: 1.7553x; 1.7553x over previous
import functools

import jax
import jax.numpy as jnp
from jax.experimental import pallas as pl


def _finalize_body(x_ref, b_ref, o_ref):
    o_ref[...] = jnp.maximum(x_ref[...] + b_ref[...], 0.0)


def _finalize(x, b):
    N, D = x.shape
    blk = 2000
    return pl.pallas_call(
        _finalize_body,
        grid=(N // blk,),
        in_specs=[
            pl.BlockSpec((blk, D), lambda i: (i, 0)),
            pl.BlockSpec((1, D), lambda i: (0, 0)),
        ],
        out_specs=pl.BlockSpec((blk, D), lambda i: (i, 0)),
        out_shape=jax.ShapeDtypeStruct((N, D), x.dtype),
    )(x, b[None, :])


def _gat(x, ei, ea, W, We, a_s, a_d, a_e, b):
    N = x.shape[0]
    src, dst = ei[0], ei[1]
    aevec = We @ a_e
    e = ea @ aevec
    h = x @ W
    s = h @ a_s
    dv = h @ a_d
    ones = jnp.ones(src.shape[0], x.dtype)
    cnt = jax.ops.segment_sum(ones, dst, num_segments=N)
    ssum = jax.ops.segment_sum(e, dst, num_segments=N)
    alpha = s[src] + dv[dst] + e
    alpha = jnp.where(alpha > 0, alpha, 0.2 * alpha)
    ex = jnp.exp(alpha)
    den = jax.ops.segment_sum(ex, dst, num_segments=N)
    a_self = s + dv + ssum / jnp.clip(cnt, 1.0)
    a_self = jnp.where(a_self > 0, a_self, 0.2 * a_self)
    exs = jnp.exp(a_self)
    num = jax.ops.segment_sum(ex[:, None] * h[src], dst, num_segments=N)
    num = num + exs[:, None] * h
    out = num / (den + exs + 1e-16)[:, None]
    return _finalize(out, b)


def kernel(x_one, edge_index_one, edge_attr_emb_one, x_two, edge_index_two,
           edge_attr_emb_two, W1, We1, as1, ad1, ae1, b1, W2, We2, as2, ad2,
           ae2, b2):
    o1 = _gat(x_one, edge_index_one, edge_attr_emb_one, W1, We1, as1, ad1, ae1, b1)
    o2 = _gat(x_two, edge_index_two, edge_attr_emb_two, W2, We2, as2, ad2, ae2, b2)
    return (o1, o2)


# speedup vs baseline: 28.3552x; 16.1538x over previous
import jax
import jax.numpy as jnp
from jax import lax
from jax.experimental import pallas as pl
from jax.experimental.pallas import tpu as pltpu
from jax.experimental.pallas import tpu_sc as plsc

N = 10000
NPAD = 10240
E = 320000
D = 128
NSUB = 16
EPT = E // NSUB
NPT = NPAD // NSUB
F32 = jnp.float32
I32 = jnp.int32


def _prep_body(x1_ref, x2_ref, w_ref, h_ref, sd_ref):
    c = pl.program_id(0)

    @pl.when(c == 0)
    def _():
        m = jnp.dot(x1_ref[...], w_ref[0], preferred_element_type=F32)
        h_ref[...] = m[:, :D]
        sd_ref[...] = m[:, D:D + 2][None]

    @pl.when(c == 1)
    def _():
        m = jnp.dot(x2_ref[...], w_ref[0], preferred_element_type=F32)
        h_ref[...] = m[:, :D]
        sd_ref[...] = m[:, D:D + 2][None]


def _tc_prep(x1, x2, wstack):
    return pl.pallas_call(
        _prep_body,
        grid=(2, NPAD // 1024),
        in_specs=[
            pl.BlockSpec((1024, D), lambda c, r: (r, 0)),
            pl.BlockSpec((1024, D), lambda c, r: (r, 0)),
            pl.BlockSpec((1, D, 2 * D), lambda c, r: (c, 0, 0)),
        ],
        out_specs=[
            pl.BlockSpec((1024, D), lambda c, r: (c * 10 + r, 0)),
            pl.BlockSpec((1, 1024, 2), lambda c, r: (c, r, 0)),
        ],
        out_shape=[
            jax.ShapeDtypeStruct((2 * NPAD, D), F32),
            jax.ShapeDtypeStruct((2, NPAD, 2), F32),
        ],
    )(x1, x2, wstack)


def _escore_body(e1_ref, e2_ref, m_ref, o_ref):
    c = pl.program_id(0)

    @pl.when(c == 0)
    def _():
        o_ref[...] = jnp.dot(e1_ref[...], m_ref[0],
                             preferred_element_type=F32)[None]

    @pl.when(c == 1)
    def _():
        o_ref[...] = jnp.dot(e2_ref[...], m_ref[0],
                             preferred_element_type=F32)[None]


def _tc_escore(ea1, ea2, mstack):
    return pl.pallas_call(
        _escore_body,
        grid=(2, 25),
        in_specs=[
            pl.BlockSpec((200, 1024), lambda c, r: (r, 0)),
            pl.BlockSpec((200, 1024), lambda c, r: (r, 0)),
            pl.BlockSpec((1, 1024, 64), lambda c, r: (c, 0, 0)),
        ],
        out_specs=pl.BlockSpec((1, 200, 64), lambda c, r: (c, r, 0)),
        out_shape=jax.ShapeDtypeStruct((2, 5000, 64), F32),
    )(ea1, ea2, mstack)


NA = 10080
K1 = 400
K3 = 80


def _sc_body(src_hbm, dst_hbm, e_hbm, h_hbm, sd_hbm, bias_hbm,
             out_hbm,
             sd2, srcb, dstb1, eb1, dst2d, eb2d, exb2d, onesb,
             srcQ, dstQ, eQ, gidx, dstS, exbQ, rows,
             denb, cntb, ssb, exsb, bbuf,
             out_acc, den_sh, cnt_sh, ssum_sh,
             semL1, semS1, semLa, semLb, semGa, semGb, semSa, semSb):
    c = lax.axis_index("c")
    s = lax.axis_index("s")
    iota = lax.iota(I32, 16)
    ebase = c * E + s * EPT

    pltpu.sync_copy(bias_hbm.at[pl.ds(c * D, D)], bbuf)
    pltpu.sync_copy(sd_hbm.at[pl.ds(c * 2 * NPAD, 2 * NPAD)], sd2)

    def _ones(v, carry):
        onesb[pl.ds(v * 16, 16)] = jnp.ones((16,), F32)
        return carry
    lax.fori_loop(0, 5, _ones, 0)

    def _zb(i, carry):
        denb[pl.ds(i * 16, 16)] = jnp.zeros((16,), F32)
        return carry
    lax.fori_loop(0, 10, _zb, 0)

    def _zc(q, carry):
        base_q = s * NPT + q * 160

        @pl.when(base_q < NA)
        def _():
            pltpu.sync_copy(denb, den_sh.at[pl.ds(base_q, 160)])
            pltpu.sync_copy(denb, cnt_sh.at[pl.ds(base_q, 160)])
            pltpu.sync_copy(denb, ssum_sh.at[pl.ds(base_q, 160)])
        return carry
    lax.fori_loop(0, NPT // 160, _zc, 0)
    plsc.subcore_barrier()

    def _drain1(r, carry2):
        pltpu.make_async_copy(exb2d.at[r], den_sh.at[dst2d.at[r]],
                              semS1).wait()
        pltpu.make_async_copy(onesb, cnt_sh.at[dst2d.at[r]], semS1).wait()
        pltpu.make_async_copy(eb2d.at[r], ssum_sh.at[dst2d.at[r]],
                              semS1).wait()
        return carry2

    def _p1(ch, carry):
        off = ebase + ch * K1
        pltpu.async_copy(src_hbm.at[pl.ds(off, K1)], srcb, semL1)
        pltpu.async_copy(dst_hbm.at[pl.ds(off, K1)], dstb1, semL1)
        pltpu.async_copy(e_hbm.at[pl.ds(off, K1)], eb1, semL1)

        @pl.when(ch > 0)
        def _():
            lax.fori_loop(0, K1 // 80, _drain1, 0)

        pltpu.make_async_copy(src_hbm.at[pl.ds(off, K1)], srcb, semL1).wait()
        pltpu.make_async_copy(dst_hbm.at[pl.ds(off, K1)], dstb1, semL1).wait()
        pltpu.make_async_copy(e_hbm.at[pl.ds(off, K1)], eb1, semL1).wait()

        def _vec(v, carry2):
            srcv = srcb[pl.ds(v * 16, 16)]
            dstv = dstb1[pl.ds(v * 16, 16)]
            ev = eb1[pl.ds(v * 16, 16)]
            sv = plsc.load_gather(sd2, [srcv * 2])
            dv = plsc.load_gather(sd2, [dstv * 2 + 1])
            a = sv + dv + ev
            a = jnp.where(a > 0, a, 0.2 * a)
            ex = jnp.exp(a)
            exb2d[v // 5, pl.ds((v % 5) * 16, 16)] = ex
            eb2d[v // 5, pl.ds((v % 5) * 16, 16)] = ev
            dst2d[v // 5, pl.ds((v % 5) * 16, 16)] = dstv
            return carry2
        lax.fori_loop(0, K1 // 16, _vec, 0)

        def _scat(r, carry2):
            pltpu.async_copy(exb2d.at[r], den_sh.at[dst2d.at[r]], semS1,
                             add=True)
            pltpu.async_copy(onesb, cnt_sh.at[dst2d.at[r]], semS1, add=True)
            pltpu.async_copy(eb2d.at[r], ssum_sh.at[dst2d.at[r]], semS1,
                             add=True)
            return carry2
        lax.fori_loop(0, K1 // 80, _scat, 0)
        return carry
    lax.fori_loop(0, EPT // K1, _p1, 0)
    lax.fori_loop(0, K1 // 80, _drain1, 0)
    plsc.subcore_barrier()

    def _p2(q, carry):
        base = s * NPT + q * 160
        gbase = c * NPAD + base

        @pl.when(base < NA)
        def _():
            pltpu.sync_copy(den_sh.at[pl.ds(base, 160)], denb)
            pltpu.sync_copy(cnt_sh.at[pl.ds(base, 160)], cntb)
            pltpu.sync_copy(ssum_sh.at[pl.ds(base, 160)], ssb)

            def _vec(v, carry2):
                nv = jnp.full((16,), base + v * 16, I32) + iota
                sv = plsc.load_gather(sd2, [nv * 2])
                dv = plsc.load_gather(sd2, [nv * 2 + 1])
                cv = cntb[pl.ds(v * 16, 16)]
                a = sv + dv + ssb[pl.ds(v * 16, 16)] / jnp.maximum(cv, 1.0)
                a = jnp.where(a > 0, a, 0.2 * a)
                ex = jnp.exp(a)
                exsb[pl.ds(v * 16, 16)] = ex
                denb[pl.ds(v * 16, 16)] = denb[pl.ds(v * 16, 16)] + ex
                return carry2
            lax.fori_loop(0, 10, _vec, 0)
            pltpu.sync_copy(denb, den_sh.at[pl.ds(base, 160)])

            def _rr(rr, carry2):
                @pl.when(base + rr * 80 < N)
                def _():
                    def _mk(v, carry3):
                        gv = jnp.full((16,), gbase + rr * 80 + v * 16, I32)
                        gidx[0, pl.ds(v * 16, 16)] = gv + iota
                        return carry3
                    lax.fori_loop(0, 5, _mk, 0)
                    pltpu.sync_copy(h_hbm.at[gidx.at[0]], rows.at[0])

                    def _scale(e2, carry3):
                        evec = plsc.load_gather(
                            exsb, [jnp.full((16,), rr * 80 + e2, I32)])
                        for f in range(8):
                            rows[0, e2, pl.ds(f * 16, 16)] = (
                                rows[0, e2, pl.ds(f * 16, 16)] * evec)
                        return carry3
                    lax.fori_loop(0, 80, _scale, 0)
                    pltpu.sync_copy(rows.at[0],
                                    out_acc.at[pl.ds(base + rr * 80, 80)])
                return carry2
            lax.fori_loop(0, 2, _rr, 0)
        return carry
    lax.fori_loop(0, NPT // 160, _p2, 0)
    plsc.subcore_barrier()

    NCH = EPT // K3

    def _loads(ch, p, sem):
        off = ebase + ch * K3
        pltpu.async_copy(src_hbm.at[pl.ds(off, K3)], srcQ.at[p], sem)
        pltpu.async_copy(dst_hbm.at[pl.ds(off, K3)], dstQ.at[p], sem)
        pltpu.async_copy(e_hbm.at[pl.ds(off, K3)], eQ.at[p], sem)

    def _wait_loads(ch, p, sem):
        off = ebase + ch * K3
        pltpu.make_async_copy(src_hbm.at[pl.ds(off, K3)], srcQ.at[p],
                              sem).wait()
        pltpu.make_async_copy(dst_hbm.at[pl.ds(off, K3)], dstQ.at[p],
                              sem).wait()
        pltpu.make_async_copy(e_hbm.at[pl.ds(off, K3)], eQ.at[p], sem).wait()

    def _compute(p):
        def _mk(v, carry):
            srcv = srcQ[p, pl.ds(v * 16, 16)]
            dstv = dstQ[p, pl.ds(v * 16, 16)]
            ev = eQ[p, pl.ds(v * 16, 16)]
            sv = plsc.load_gather(sd2, [srcv * 2])
            dv = plsc.load_gather(sd2, [dstv * 2 + 1])
            a = sv + dv + ev
            a = jnp.where(a > 0, a, 0.2 * a)
            exbQ[p, pl.ds(v * 16, 16)] = jnp.exp(a)
            gidx[p, pl.ds(v * 16, 16)] = srcv + c * NPAD
            return carry
        lax.fori_loop(0, 5, _mk, 0)

    def _copy_dst(p):
        def _cp(v, carry):
            dstS[p, pl.ds(v * 16, 16)] = dstQ[p, pl.ds(v * 16, 16)]
            return carry
        lax.fori_loop(0, 5, _cp, 0)

    def _scale3(p):
        def _s(e2, carry):
            evec = plsc.load_gather(exbQ.at[p], [jnp.full((16,), e2, I32)])
            for f in range(8):
                rows[p, e2, pl.ds(f * 16, 16)] = (
                    rows[p, e2, pl.ds(f * 16, 16)] * evec)
            return carry
        lax.fori_loop(0, 80, _s, 0)

    _loads(0, 0, semLa)
    _wait_loads(0, 0, semLa)
    _compute(0)
    _copy_dst(0)
    pltpu.async_copy(h_hbm.at[gidx.at[0]], rows.at[0], semGa)
    _loads(1, 1, semLb)

    def _p3(i, carry):
        ch = 2 * i

        @pl.when(ch + 1 < NCH)
        def _():
            _wait_loads(ch + 1, 1, semLb)
            _compute(1)

            @pl.when(ch > 0)
            def _():
                pltpu.make_async_copy(rows.at[1], out_acc.at[dstS.at[1]],
                                      semSb).wait()
            pltpu.async_copy(h_hbm.at[gidx.at[1]], rows.at[1], semGb)
            _copy_dst(1)

            @pl.when(ch + 2 < NCH)
            def _():
                _loads(ch + 2, 0, semLa)

        pltpu.make_async_copy(h_hbm.at[gidx.at[0]], rows.at[0], semGa).wait()
        _scale3(0)
        pltpu.async_copy(rows.at[0], out_acc.at[dstS.at[0]], semSa, add=True)

        chb = 2 * i + 1

        @pl.when(chb < NCH)
        def _():
            @pl.when(chb + 1 < NCH)
            def _():
                _wait_loads(chb + 1, 0, semLa)
                _compute(0)
                pltpu.make_async_copy(rows.at[0], out_acc.at[dstS.at[0]],
                                      semSa).wait()
                pltpu.async_copy(h_hbm.at[gidx.at[0]], rows.at[0], semGa)
                _copy_dst(0)

                @pl.when(chb + 2 < NCH)
                def _():
                    _loads(chb + 2, 1, semLb)

            pltpu.make_async_copy(h_hbm.at[gidx.at[1]], rows.at[1],
                                  semGb).wait()
            _scale3(1)
            pltpu.async_copy(rows.at[1], out_acc.at[dstS.at[1]], semSb,
                             add=True)
        return carry
    lax.fori_loop(0, (NCH + 1) // 2, _p3, 0)

    pltpu.make_async_copy(rows.at[0], out_acc.at[dstS.at[0]], semSa).wait()
    pltpu.make_async_copy(rows.at[1], out_acc.at[dstS.at[1]], semSb).wait()
    plsc.subcore_barrier()

    def _p4(q, carry):
        base = s * NPT + q * 80

        @pl.when(base < N)
        def _():
            pltpu.sync_copy(out_acc.at[pl.ds(base, 80)], rows.at[0])
            pltpu.sync_copy(den_sh.at[pl.ds(base, 80)], denb.at[pl.ds(0, 80)])

            def _row(e2, carry2):
                dvec = plsc.load_gather(denb, [jnp.full((16,), e2, I32)])
                ivec = 1.0 / (dvec + 1e-16)
                for f in range(8):
                    v = (rows[0, e2, pl.ds(f * 16, 16)] * ivec
                         + bbuf[pl.ds(f * 16, 16)])
                    rows[0, e2, pl.ds(f * 16, 16)] = jnp.maximum(v, 0.0)
                return carry2
            lax.fori_loop(0, 80, _row, 0)
            pltpu.sync_copy(rows.at[0], out_hbm.at[c, pl.ds(base, 80)])
        return carry
    lax.fori_loop(0, NPT // 80, _p4, 0)


def _sc_call(srcf, dstf, ef, h_tab, sdf, biasf):
    mesh = plsc.VectorSubcoreMesh(core_axis_name="c", subcore_axis_name="s")
    f = pl.kernel(
        _sc_body,
        out_type=jax.ShapeDtypeStruct((2, N, D), F32),
        mesh=mesh,
        compiler_params=pltpu.CompilerParams(needs_layout_passes=False),
        scratch_types=[
            pltpu.VMEM((2 * NPAD,), F32),
            pltpu.VMEM((K1,), I32),
            pltpu.VMEM((K1,), I32),
            pltpu.VMEM((K1,), F32),
            pltpu.VMEM((K1 // 80, 80), I32),
            pltpu.VMEM((K1 // 80, 80), F32),
            pltpu.VMEM((K1 // 80, 80), F32),
            pltpu.VMEM((80,), F32),
            pltpu.VMEM((2, K3), I32),
            pltpu.VMEM((2, K3), I32),
            pltpu.VMEM((2, K3), F32),
            pltpu.VMEM((2, K3), I32),
            pltpu.VMEM((2, K3), I32),
            pltpu.VMEM((2, K3), F32),
            pltpu.VMEM((2, K3, D), F32),
            pltpu.VMEM((160,), F32),
            pltpu.VMEM((160,), F32),
            pltpu.VMEM((160,), F32),
            pltpu.VMEM((160,), F32),
            pltpu.VMEM((D,), F32),
            pltpu.VMEM_SHARED((N, D), F32),
            pltpu.VMEM_SHARED((NA,), F32),
            pltpu.VMEM_SHARED((NA,), F32),
            pltpu.VMEM_SHARED((NA,), F32),
            pltpu.SemaphoreType.DMA,
            pltpu.SemaphoreType.DMA,
            pltpu.SemaphoreType.DMA,
            pltpu.SemaphoreType.DMA,
            pltpu.SemaphoreType.DMA,
            pltpu.SemaphoreType.DMA,
            pltpu.SemaphoreType.DMA,
            pltpu.SemaphoreType.DMA,
        ],
    )
    return f(srcf, dstf, ef, h_tab, sdf, biasf)


def kernel(x_one, edge_index_one, edge_attr_emb_one, x_two, edge_index_two,
           edge_attr_emb_two, W1, We1, as1, ad1, ae1, b1, W2, We2, as2, ad2,
           ae2, b2):
    wf1 = jnp.concatenate([W1, (W1 @ as1)[:, None], (W1 @ ad1)[:, None],
                           jnp.zeros((D, D - 2), F32)], axis=1)
    wf2 = jnp.concatenate([W2, (W2 @ as2)[:, None], (W2 @ ad2)[:, None],
                           jnp.zeros((D, D - 2), F32)], axis=1)
    wstack = jnp.stack([wf1, wf2])
    h_tab, sd_out = _tc_prep(x_one, x_two, wstack)

    eye = jnp.eye(64, dtype=F32)
    m1 = jnp.kron(eye, (We1 @ ae1)[:, None])
    m2 = jnp.kron(eye, (We2 @ ae2)[:, None])
    mstack = jnp.stack([m1, m2])
    ef = _tc_escore(edge_attr_emb_one.reshape(5000, 1024),
                    edge_attr_emb_two.reshape(5000, 1024),
                    mstack)

    srcf = jnp.stack([edge_index_one[0], edge_index_two[0]]).reshape(-1)
    dstf = jnp.stack([edge_index_one[1], edge_index_two[1]]).reshape(-1)
    ef_flat = ef.reshape(-1)
    sdf = sd_out.reshape(-1)
    biasf = jnp.concatenate([b1, b2])

    out = _sc_call(srcf, dstf, ef_flat, h_tab, sdf, biasf)
    return (out[0], out[1])
